# baseline (speedup 1.0000x reference)
import functools

import jax
import jax.numpy as jnp
from jax import lax
from jax.experimental import pallas as pl
from jax.experimental.pallas import tpu as pltpu
from jax.experimental.pallas import tpu_sc as plsc

_VOCAB = 1000000
_D = 32
_B = 4096
_NC = 2
_NS = 16
_NW = _NC * _NS
_BPW = _B // _NW
_CH = 32

_PACK = 128 // _D
_VPACK = _VOCAB // _PACK
_RB = 2000

_ROW_BLOCK = 512


_RSTEPS = _VPACK // _RB


def _repack_body(in0, in1, in2, in3, out_ref):
    out_ref[...] = jnp.concatenate(
        [in0[...], in1[...], in2[...], in3[...]], axis=1)


_repack_tc = pl.pallas_call(
    _repack_body,
    grid=(_RSTEPS,),
    in_specs=[
        pl.BlockSpec((_RB, _D), lambda i, k=k: (i + k * _RSTEPS, 0))
        for k in range(_PACK)
    ],
    out_specs=pl.BlockSpec((_RB, 128), lambda i: (i, 0)),
    out_shape=jax.ShapeDtypeStruct((_VPACK, 128), jnp.float32),
)


@functools.partial(
    pl.kernel,
    out_type=(
        jax.ShapeDtypeStruct((_B, 128), jnp.float32),
        jax.ShapeDtypeStruct((_B, 128), jnp.float32),
    ),
    mesh=plsc.VectorSubcoreMesh(core_axis_name="c", subcore_axis_name="s"),
    scratch_types=(
        pltpu.VMEM((_BPW,), jnp.int32),
        pltpu.VMEM((_BPW, 128), jnp.float32),
        pltpu.SemaphoreType.DMA,
    ),
)
def _gather_sc(emb_hbm, tgt_hbm, ctx_hbm, out_t_hbm, out_c_hbm,
               idx_v, out_v, sem):
    wid = lax.axis_index("s") * _NC + lax.axis_index("c")
    base = wid * _BPW

    for idx_hbm, out_hbm in ((tgt_hbm, out_t_hbm), (ctx_hbm, out_c_hbm)):
        pltpu.sync_copy(idx_hbm.at[pl.ds(base, _BPW)], idx_v)

        def _fire(r):
            cps = []
            for g in range(_CH // 16):
                gbase = r * _CH + g * 16
                vec = idx_v[pl.ds(gbase, 16)]
                for l in range(16):
                    v = vec[l]
                    cps.append(pltpu.async_copy(
                        emb_hbm.at[v], out_v.at[gbase + l], sem))
            return cps

        pending = _fire(0)
        for r in range(_BPW // _CH):
            nxt = _fire(r + 1) if r + 1 < _BPW // _CH else []
            for cp in pending:
                cp.wait()
            pending = nxt

        pltpu.sync_copy(out_v, out_hbm.at[pl.ds(base, _BPW)])


def _select_rows(raw, mod):
    out = jnp.zeros((raw.shape[0], _D), jnp.float32)
    for k in range(_PACK):
        out = out + jnp.where(mod == k, raw[:, k * _D:(k + 1) * _D], 0.0)
    return out


def _scores_body(t_ref, c_ref, tm_ref, cm_ref, o_ref):
    t = _select_rows(t_ref[...], tm_ref[...])
    c = _select_rows(c_ref[...], cm_ref[...])
    o_ref[...] = lax.dot_general(
        t, c,
        dimension_numbers=(((1,), (1,)), ((), ())),
        preferred_element_type=jnp.float32,
    )


_scores_tc = pl.pallas_call(
    _scores_body,
    grid=(_B // _ROW_BLOCK,),
    in_specs=[
        pl.BlockSpec((_ROW_BLOCK, 128), lambda i: (i, 0)),
        pl.BlockSpec((_B, 128), lambda i: (0, 0)),
        pl.BlockSpec((_ROW_BLOCK, 1), lambda i: (i, 0)),
        pl.BlockSpec((_B, 1), lambda i: (0, 0)),
    ],
    out_specs=pl.BlockSpec((_ROW_BLOCK, _B), lambda i: (i, 0)),
    out_shape=jax.ShapeDtypeStruct((_B, _B), jnp.float32),
)


def kernel(target, context, embeddings):
    target = target.astype(jnp.int32)
    context = context.astype(jnp.int32)
    packed = _repack_tc(embeddings, embeddings, embeddings, embeddings)
    tgt_raw, ctx_raw = _gather_sc(packed, target % _VPACK, context % _VPACK)
    tmod = (target // _VPACK).reshape(_B, 1)
    cmod = (context // _VPACK).reshape(_B, 1)
    return _scores_tc(tgt_raw, ctx_raw, tmod, cmod)


# speedup vs baseline: 2.7636x; 2.7636x over previous
import functools

import jax
import jax.numpy as jnp
from jax import lax
from jax.experimental import pallas as pl
from jax.experimental.pallas import tpu as pltpu
from jax.experimental.pallas import tpu_sc as plsc

_VOCAB = 1000000
_D = 32
_B = 4096
_NC = 2
_NS = 16
_NW = _NC * _NS
_BPW = _B // _NW
_CH = 32

_ROW_BLOCK = 512


@functools.partial(
    pl.kernel,
    out_type=(
        jax.ShapeDtypeStruct((_B, _D), jnp.float32),
        jax.ShapeDtypeStruct((_B, _D), jnp.float32),
    ),
    mesh=plsc.VectorSubcoreMesh(core_axis_name="c", subcore_axis_name="s"),
    scratch_types=(
        pltpu.VMEM((_BPW,), jnp.int32),
        pltpu.VMEM((_BPW, _D), jnp.float32),
        pltpu.SemaphoreType.DMA,
    ),
)
def _gather_sc(emb_hbm, tgt_hbm, ctx_hbm, out_t_hbm, out_c_hbm,
               idx_v, out_v, sem):
    wid = lax.axis_index("s") * _NC + lax.axis_index("c")
    base = wid * _BPW

    for idx_hbm, out_hbm in ((tgt_hbm, out_t_hbm), (ctx_hbm, out_c_hbm)):
        pltpu.sync_copy(idx_hbm.at[pl.ds(base, _BPW)], idx_v)

        def _fire(r):
            cps = []
            for g in range(_CH // 16):
                gbase = r * _CH + g * 16
                vec = idx_v[pl.ds(gbase, 16)]
                for l in range(16):
                    v = vec[l]
                    cps.append(pltpu.async_copy(
                        emb_hbm.at[v >> 3, v & 7], out_v.at[gbase + l], sem))
            return cps

        pending = _fire(0)
        for r in range(_BPW // _CH):
            nxt = _fire(r + 1) if r + 1 < _BPW // _CH else []
            for cp in pending:
                cp.wait()
            pending = nxt

        pltpu.sync_copy(out_v, out_hbm.at[pl.ds(base, _BPW)])


def _scores_body(t_ref, c_ref, o_ref):
    o_ref[...] = lax.dot_general(
        t_ref[...], c_ref[...],
        dimension_numbers=(((1,), (1,)), ((), ())),
        preferred_element_type=jnp.float32,
    )


_scores_tc = pl.pallas_call(
    _scores_body,
    grid=(_B // _ROW_BLOCK,),
    in_specs=[
        pl.BlockSpec((_ROW_BLOCK, _D), lambda i: (i, 0)),
        pl.BlockSpec((_B, _D), lambda i: (0, 0)),
    ],
    out_specs=pl.BlockSpec((_ROW_BLOCK, _B), lambda i: (i, 0)),
    out_shape=jax.ShapeDtypeStruct((_B, _B), jnp.float32),
)


def kernel(target, context, embeddings):
    emb3 = embeddings.reshape(_VOCAB // 8, 8, _D)
    tgt_rows, ctx_rows = _gather_sc(
        emb3, target.astype(jnp.int32), context.astype(jnp.int32))
    return _scores_tc(tgt_rows, ctx_rows)
